# Initial kernel scaffold; baseline (speedup 1.0000x reference)
import jax
import jax.numpy as jnp
from jax.experimental import pallas as pl


def kernel(x):
    raise NotImplementedError("write your pallas kernel here")


# speedup vs baseline: 1.4839x; 1.4839x over previous
import numpy as np
import jax
import jax.numpy as jnp
from jax import lax
from jax.experimental import pallas as pl
from jax.experimental.pallas import tpu as pltpu

_W, _H, _SX, _SY, _R = 32, 32, 4, 4, 512


def _partition():
    hsy, wsx = _H // _SY, _W // _SX
    buf = np.zeros((hsy, wsx, _SY * _SX), dtype=np.int64)
    buf[..., 0] = -1
    buf = buf.reshape(hsy, wsx, _SY, _SX).transpose(0, 2, 1, 3).reshape(hsy * _SY, wsx * _SX)
    order = np.argsort(buf.reshape(-1), kind="stable")
    num_dst = hsy * wsx
    return order[:num_dst], order[num_dst:], num_dst


_B_IDX, _A_IDX, _NUM_DST = _partition()
_N = _W * _H
_NUM_SRC = _N - _NUM_DST
_R_EFF = min(_NUM_SRC, _R)
_NUM_UNM = _NUM_SRC - _R_EFF
_NOUT = _NUM_UNM + _NUM_DST


def _consts():
    is_dst = np.zeros((_N,), np.float32)
    is_dst[_B_IDX] = 1.0
    dtar = np.zeros((_N,), np.float32)
    dtar[_B_IDX] = _NUM_UNM + np.arange(_NUM_DST, dtype=np.float32)
    onehot_b = np.zeros((_NUM_DST, _N), np.float32)
    onehot_b[np.arange(_NUM_DST), _B_IDX] = 1.0
    return is_dst, dtar, onehot_b


_IS_DST_NP, _DTAR_NP, _ONEHOT_B_NP = _consts()


def _body(x_ref, nsq_ref, ohb_ref, isdst_row_ref, srccol_ref, dtar_row_ref, o_ref):
    N, C = _N, x_ref.shape[2]
    x = x_ref[0]
    nsq = nsq_ref[0]
    nrm = jnp.where(nsq == 0.0, 0.0, nsq * lax.rsqrt(nsq))
    met = x * pl.reciprocal(nrm, approx=True)

    ohb = ohb_ref[...]
    bmet = lax.dot_general(ohb, met, (((1,), (0,)), ((), ())),
                           preferred_element_type=jnp.float32,
                           precision=lax.Precision.HIGHEST)
    scores = lax.dot_general(bmet, met, (((1,), (1,)), ((), ())),
                             preferred_element_type=jnp.float32)
    v_row = jnp.max(scores, axis=0, keepdims=True)
    d_iota = lax.broadcasted_iota(jnp.int32, scores.shape, 0)
    nidx_row = jnp.min(jnp.where(scores == v_row, d_iota, 10 ** 9),
                       axis=0, keepdims=True).astype(jnp.float32)

    v_col = jnp.reshape(v_row, (N, 1))
    srccol = srccol_ref[...]

    CH = 256
    rank_row = jnp.zeros((1, N), jnp.float32)
    for c in range(0, N, CH):
        vc = v_col[c:c + CH]
        sc = srccol[c:c + CH]
        ti = lax.broadcasted_iota(jnp.int32, (CH, N), 1)
        tpi = c + lax.broadcasted_iota(jnp.int32, (CH, N), 0)
        ahead = (vc > v_row) | ((vc == v_row) & (tpi < ti))
        contrib = jnp.where(ahead, sc, 0.0)
        rank_row = rank_row + jnp.sum(contrib, axis=0, keepdims=True)

    isdst_row = isdst_row_ref[...]
    dtar_row = dtar_row_ref[...]
    merged = rank_row < float(_R_EFF)
    tgt_row = jnp.where(isdst_row > 0, dtar_row,
                        jnp.where(merged, float(_NUM_UNM) + nidx_row,
                                  rank_row - float(_R_EFF)))

    k_iota = lax.broadcasted_iota(jnp.int32, (_NOUT, N), 0)
    m0 = jnp.where(k_iota == tgt_row.astype(jnp.int32), 1.0, 0.0)
    num = lax.dot_general(m0, x, (((1,), (0,)), ((), ())),
                          preferred_element_type=jnp.float32,
                          precision=lax.Precision.HIGHEST)
    den = jnp.sum(m0, axis=1, keepdims=True)
    o_ref[0] = num / den


def kernel(x):
    B, N, C = x.shape
    assert N == _N
    nsq = jnp.sum(x * x, axis=-1, keepdims=True)
    ohb = jnp.asarray(_ONEHOT_B_NP)
    isdst_row = jnp.asarray(_IS_DST_NP.reshape(1, _N))
    srccol = jnp.asarray((1.0 - _IS_DST_NP).reshape(_N, 1))
    dtar_row = jnp.asarray(_DTAR_NP.reshape(1, _N))

    return pl.pallas_call(
        _body,
        grid=(B,),
        in_specs=[
            pl.BlockSpec((1, N, C), lambda i: (i, 0, 0)),
            pl.BlockSpec((1, N, 1), lambda i: (i, 0, 0)),
            pl.BlockSpec((_NUM_DST, N), lambda i: (0, 0)),
            pl.BlockSpec((1, N), lambda i: (0, 0)),
            pl.BlockSpec((N, 1), lambda i: (0, 0)),
            pl.BlockSpec((1, N), lambda i: (0, 0)),
        ],
        out_specs=pl.BlockSpec((1, _NOUT, C), lambda i: (i, 0, 0)),
        out_shape=jax.ShapeDtypeStruct((B, _NOUT, C), x.dtype),
        compiler_params=pltpu.CompilerParams(
            dimension_semantics=("arbitrary",),
        ),
    )(x, nsq, ohb, isdst_row, srccol, dtar_row)


# speedup vs baseline: 2.4914x; 1.6790x over previous
import numpy as np
import jax
import jax.numpy as jnp
from jax import lax
from jax.experimental import pallas as pl
from jax.experimental.pallas import tpu as pltpu

_W, _H, _SX, _SY, _R = 32, 32, 4, 4, 512


def _partition():
    hsy, wsx = _H // _SY, _W // _SX
    buf = np.zeros((hsy, wsx, _SY * _SX), dtype=np.int64)
    buf[..., 0] = -1
    buf = buf.reshape(hsy, wsx, _SY, _SX).transpose(0, 2, 1, 3).reshape(hsy * _SY, wsx * _SX)
    order = np.argsort(buf.reshape(-1), kind="stable")
    num_dst = hsy * wsx
    return order[:num_dst], order[num_dst:], num_dst


_B_IDX, _A_IDX, _NUM_DST = _partition()
_N = _W * _H
_NUM_SRC = _N - _NUM_DST
_R_EFF = min(_NUM_SRC, _R)
_NUM_UNM = _NUM_SRC - _R_EFF
_NOUT = _NUM_UNM + _NUM_DST


def _consts():
    is_dst = np.zeros((_N,), np.float32)
    is_dst[_B_IDX] = 1.0
    dtar = np.zeros((_N,), np.float32)
    dtar[_B_IDX] = _NUM_UNM + np.arange(_NUM_DST, dtype=np.float32)
    onehot_b = np.zeros((_NUM_DST, _N), np.float32)
    onehot_b[np.arange(_NUM_DST), _B_IDX] = 1.0
    return is_dst, dtar, onehot_b


_IS_DST_NP, _DTAR_NP, _ONEHOT_B_NP = _consts()


def _body(x_ref, nsq_ref, isdst_row_ref, srccol_ref, dtar_row_ref, o_ref):
    N, C = _N, x_ref.shape[2]
    x = x_ref[0]
    nsq = nsq_ref[0]
    nrm = jnp.where(nsq == 0.0, 0.0, nsq * lax.rsqrt(nsq))
    met = x * pl.reciprocal(nrm, approx=True)

    bmet = met.reshape(8, 4, 8, 4, C)[:, 0, :, 0, :].reshape(_NUM_DST, C)
    scores = lax.dot_general(bmet, met, (((1,), (1,)), ((), ())),
                             preferred_element_type=jnp.float32)
    v_row = jnp.max(scores, axis=0, keepdims=True)
    d_iota = lax.broadcasted_iota(jnp.int32, scores.shape, 0)
    nidx_row = jnp.min(jnp.where(scores == v_row, d_iota, 10 ** 9),
                       axis=0, keepdims=True).astype(jnp.float32)

    v_col = jnp.reshape(v_row, (N, 1))
    srccol = srccol_ref[...]

    CH = 256
    rank_row = jnp.zeros((1, N), jnp.float32)
    for c in range(0, N, CH):
        vc = v_col[c:c + CH]
        sc = srccol[c:c + CH]
        ti = lax.broadcasted_iota(jnp.int32, (CH, N), 1)
        tpi = c + lax.broadcasted_iota(jnp.int32, (CH, N), 0)
        ahead = (vc > v_row) | ((vc == v_row) & (tpi < ti))
        contrib = jnp.where(ahead, sc, 0.0)
        rank_row = rank_row + jnp.sum(contrib, axis=0, keepdims=True)

    isdst_row = isdst_row_ref[...]
    dtar_row = dtar_row_ref[...]
    merged = rank_row < float(_R_EFF)
    tgt_row = jnp.where(isdst_row > 0, dtar_row,
                        jnp.where(merged, float(_NUM_UNM) + nidx_row,
                                  rank_row - float(_R_EFF)))

    k_iota = lax.broadcasted_iota(jnp.int32, (_NOUT, N), 0)
    m0 = jnp.where(k_iota == tgt_row.astype(jnp.int32), 1.0, 0.0)
    num = lax.dot_general(m0, x, (((1,), (0,)), ((), ())),
                          preferred_element_type=jnp.float32)
    den = jnp.sum(m0, axis=1, keepdims=True)
    o_ref[0] = num / den


def kernel(x):
    B, N, C = x.shape
    assert N == _N
    nsq = jnp.sum(x * x, axis=-1, keepdims=True)
    isdst_row = jnp.asarray(_IS_DST_NP.reshape(1, _N))
    srccol = jnp.asarray((1.0 - _IS_DST_NP).reshape(_N, 1))
    dtar_row = jnp.asarray(_DTAR_NP.reshape(1, _N))

    return pl.pallas_call(
        _body,
        grid=(B,),
        in_specs=[
            pl.BlockSpec((1, N, C), lambda i: (i, 0, 0)),
            pl.BlockSpec((1, N, 1), lambda i: (i, 0, 0)),
            pl.BlockSpec((1, N), lambda i: (0, 0)),
            pl.BlockSpec((N, 1), lambda i: (0, 0)),
            pl.BlockSpec((1, N), lambda i: (0, 0)),
        ],
        out_specs=pl.BlockSpec((1, _NOUT, C), lambda i: (i, 0, 0)),
        out_shape=jax.ShapeDtypeStruct((B, _NOUT, C), x.dtype),
        compiler_params=pltpu.CompilerParams(
            dimension_semantics=("arbitrary",),
        ),
    )(x, nsq, isdst_row, srccol, dtar_row)
